# baseline (speedup 1.0000x reference)
import functools

import jax
import jax.numpy as jnp
from jax import lax
from jax.experimental import pallas as pl
from jax.experimental.pallas import tpu as pltpu

_PRE_NMS_TOP_K = 200
_MAX_DET = 100
_IOU_THR = 0.5
_SCORE_THR = 0.05
_K = 256
_OUT_LANES = 128


def _body(sc_ref, co_ref, os_ref, oy1_ref, ox1_ref, oy2_ref, ox2_ref,
          oc_ref, ov_ref, *, C, NP):
    s = sc_ref[0]
    s = jnp.where(s > _SCORE_THR, s, -1.0)
    co = co_ref[0]
    y1b = co[0:1, :]
    x1b = co[1:2, :]
    y2b = co[2:3, :]
    x2b = co[3:4, :]

    iota_n = lax.broadcasted_iota(jnp.int32, (C, NP), 1)
    slot_iota = lax.broadcasted_iota(jnp.int32, (C, _K), 1)

    def sel_body(j, carry):
        s, ts, ty1, tx1, ty2, tx2 = carry
        m = jnp.max(s, axis=1, keepdims=True)
        cand = jnp.where(s == m, iota_n, jnp.int32(1 << 30))
        idx = jnp.min(cand, axis=1, keepdims=True)
        upd = iota_n == idx
        updf = upd.astype(jnp.float32)
        vy1 = jnp.sum(updf * y1b, axis=1, keepdims=True)
        vx1 = jnp.sum(updf * x1b, axis=1, keepdims=True)
        vy2 = jnp.sum(updf * y2b, axis=1, keepdims=True)
        vx2 = jnp.sum(updf * x2b, axis=1, keepdims=True)
        s = jnp.where(upd, -2.0, s)
        slotm = slot_iota == j
        ts = jnp.where(slotm, m, ts)
        ty1 = jnp.where(slotm, vy1, ty1)
        tx1 = jnp.where(slotm, vx1, tx1)
        ty2 = jnp.where(slotm, vy2, ty2)
        tx2 = jnp.where(slotm, vx2, tx2)
        return (s, ts, ty1, tx1, ty2, tx2)

    init = (
        s,
        jnp.full((C, _K), -3.0, jnp.float32),
        jnp.zeros((C, _K), jnp.float32),
        jnp.zeros((C, _K), jnp.float32),
        jnp.zeros((C, _K), jnp.float32),
        jnp.zeros((C, _K), jnp.float32),
    )
    _, ts, ty1, tx1, ty2, tx2 = lax.fori_loop(0, _PRE_NMS_TOP_K, sel_body, init)

    area = (ty2 - ty1) * (tx2 - tx1)
    keep = (ts > _SCORE_THR).astype(jnp.int32)

    def nms_body(i, keep):
        kb = keep > 0
        im = slot_iota == i
        imf = im.astype(jnp.float32)
        byi1 = jnp.sum(imf * ty1, axis=1, keepdims=True)
        bxi1 = jnp.sum(imf * tx1, axis=1, keepdims=True)
        byi2 = jnp.sum(imf * ty2, axis=1, keepdims=True)
        bxi2 = jnp.sum(imf * tx2, axis=1, keepdims=True)
        ai = (byi2 - byi1) * (bxi2 - bxi1)
        ki = jnp.sum(jnp.where(im & kb, 1, 0), axis=1, keepdims=True) > 0
        yy1 = jnp.maximum(byi1, ty1)
        xx1 = jnp.maximum(bxi1, tx1)
        yy2 = jnp.minimum(byi2, ty2)
        xx2 = jnp.minimum(bxi2, tx2)
        inter = jnp.maximum(yy2 - yy1, 0.0) * jnp.maximum(xx2 - xx1, 0.0)
        union = ai + area - inter
        iou = inter / jnp.maximum(union, 1e-8)
        sup = (iou > _IOU_THR) & (slot_iota > i) & ki
        return jnp.where(sup, 0, keep)

    keep = lax.fori_loop(0, _PRE_NMS_TOP_K, nms_body, keep)
    fs = jnp.where(keep > 0, ts, -1.0)

    gflat = lax.broadcasted_iota(jnp.int32, (C, _K), 0) * _K + slot_iota
    lane_iota = lax.broadcasted_iota(jnp.int32, (1, _OUT_LANES), 1)

    def mrg_body(i, carry):
        fs, os_, oy1, ox1, oy2, ox2, oc = carry
        m = jnp.max(fs)
        cand = jnp.where(fs == m, gflat, jnp.int32(1 << 30))
        gidx = jnp.min(cand)
        upd = gflat == gidx
        updf = upd.astype(jnp.float32)
        gy1 = jnp.sum(updf * ty1)
        gx1 = jnp.sum(updf * tx1)
        gy2 = jnp.sum(updf * ty2)
        gx2 = jnp.sum(updf * tx2)
        ccls = gidx // _K
        fs = jnp.where(upd, -4.0, fs)
        om = lane_iota == i
        os_ = jnp.where(om, m, os_)
        oy1 = jnp.where(om, gy1, oy1)
        ox1 = jnp.where(om, gx1, ox1)
        oy2 = jnp.where(om, gy2, oy2)
        ox2 = jnp.where(om, gx2, ox2)
        oc = jnp.where(om, ccls, oc)
        return (fs, os_, oy1, ox1, oy2, ox2, oc)

    minit = (
        fs,
        jnp.full((1, _OUT_LANES), -9.0, jnp.float32),
        jnp.zeros((1, _OUT_LANES), jnp.float32),
        jnp.zeros((1, _OUT_LANES), jnp.float32),
        jnp.zeros((1, _OUT_LANES), jnp.float32),
        jnp.zeros((1, _OUT_LANES), jnp.float32),
        jnp.zeros((1, _OUT_LANES), jnp.int32),
    )
    _, os_, oy1, ox1, oy2, ox2, oc = lax.fori_loop(0, _MAX_DET, mrg_body, minit)

    vm = os_ > _SCORE_THR
    ov_ref[0] = jnp.sum(vm.astype(jnp.int32), axis=1, keepdims=True)
    os_ref[0] = jnp.where(vm, os_, 0.0)
    oy1_ref[0] = jnp.where(vm, oy1, 0.0)
    ox1_ref[0] = jnp.where(vm, ox1, 0.0)
    oy2_ref[0] = jnp.where(vm, oy2, 0.0)
    ox2_ref[0] = jnp.where(vm, ox2, 0.0)
    oc_ref[0] = jnp.where(vm, oc, 0)


@jax.jit
def kernel(boxes, scores):
    B, N, C = scores.shape
    NP = ((N + 127) // 128) * 128

    b = boxes[:, :, 0, :]
    co = jnp.transpose(b, (0, 2, 1))
    co = jnp.pad(co, ((0, 0), (0, 0), (0, NP - N)))
    sc = jnp.transpose(scores, (0, 2, 1))
    sc = jnp.pad(sc, ((0, 0), (0, 0), (0, NP - N)), constant_values=-2.0)

    out = pl.pallas_call(
        functools.partial(_body, C=C, NP=NP),
        grid=(B,),
        in_specs=[
            pl.BlockSpec((1, C, NP), lambda i: (i, 0, 0)),
            pl.BlockSpec((1, 4, NP), lambda i: (i, 0, 0)),
        ],
        out_specs=[
            pl.BlockSpec((1, 1, _OUT_LANES), lambda i: (i, 0, 0)),
            pl.BlockSpec((1, 1, _OUT_LANES), lambda i: (i, 0, 0)),
            pl.BlockSpec((1, 1, _OUT_LANES), lambda i: (i, 0, 0)),
            pl.BlockSpec((1, 1, _OUT_LANES), lambda i: (i, 0, 0)),
            pl.BlockSpec((1, 1, _OUT_LANES), lambda i: (i, 0, 0)),
            pl.BlockSpec((1, 1, _OUT_LANES), lambda i: (i, 0, 0)),
            pl.BlockSpec((1, 1, 1), lambda i: (i, 0, 0)),
        ],
        out_shape=[
            jax.ShapeDtypeStruct((B, 1, _OUT_LANES), jnp.float32),
            jax.ShapeDtypeStruct((B, 1, _OUT_LANES), jnp.float32),
            jax.ShapeDtypeStruct((B, 1, _OUT_LANES), jnp.float32),
            jax.ShapeDtypeStruct((B, 1, _OUT_LANES), jnp.float32),
            jax.ShapeDtypeStruct((B, 1, _OUT_LANES), jnp.float32),
            jax.ShapeDtypeStruct((B, 1, _OUT_LANES), jnp.int32),
            jax.ShapeDtypeStruct((B, 1, 1), jnp.int32),
        ],
    )(sc, co)
    os_, oy1, ox1, oy2, ox2, oc, ov = (o[:, 0] for o in out)

    nms_boxes = jnp.stack(
        [oy1[:, :_MAX_DET], ox1[:, :_MAX_DET], oy2[:, :_MAX_DET], ox2[:, :_MAX_DET]],
        axis=-1,
    )
    return nms_boxes, os_[:, :_MAX_DET], oc[:, :_MAX_DET], ov[:, 0]
